# Initial kernel scaffold; baseline (speedup 1.0000x reference)
import jax
import jax.numpy as jnp
from jax.experimental import pallas as pl


def kernel(x, edge_index, W1_0, b1_0, gamma_0, beta_0, W2_0, b2_0, W1_1, b1_1, gamma_1, beta_1, W2_1, b2_1):
    raise NotImplementedError("write your pallas kernel here")


# speedup vs baseline: 6.0887x; 6.0887x over previous
import functools

import jax
import jax.numpy as jnp
from jax import lax
from jax.experimental import pallas as pl
from jax.experimental.pallas import tpu as pltpu
from jax.experimental.pallas import tpu_sc as plsc

NN = 10000
DD = 128
HH = 256
EE = 320000
EPSM = 1e-7
SHIFT = 30.0

NCORE = 2
NSUB = 16
EPT = EE // NSUB
CHUNK = 80
NCHUNK = EPT // CHUNK
NP = 10112
RPT = NP // NSUB

BLK = 2000
NBLK = NN // BLK


def _sc_edge_body(t0_hbm, t1_hbm, src_hbm, dst_hbm, zeros_hbm, out_hbm,
                  acc, src_v, dst_v, rows_v, sem):
    c = lax.axis_index("c")
    s = lax.axis_index("s")
    stripe = pl.ds(s * RPT, RPT)

    pltpu.sync_copy(zeros_hbm, acc.at[stripe])
    plsc.subcore_barrier()

    def run_edges(tbl):
        def chunk(j, carry):
            off = s * EPT + j * CHUNK
            pltpu.sync_copy(src_hbm.at[pl.ds(off, CHUNK)], src_v)
            pltpu.sync_copy(dst_hbm.at[pl.ds(off, CHUNK)], dst_v)
            pltpu.async_copy(tbl.at[src_v], rows_v, sem).wait()
            pltpu.sync_copy(rows_v, acc.at[dst_v], add=True)
            return carry
        lax.fori_loop(0, NCHUNK, chunk, 0)

    @pl.when(c == 0)
    def _():
        run_edges(t0_hbm)

    @pl.when(c == 1)
    def _():
        run_edges(t1_hbm)

    plsc.subcore_barrier()

    @pl.when(c == 0)
    def _():
        pltpu.sync_copy(acc.at[stripe], out_hbm.at[0, stripe])

    @pl.when(c == 1)
    def _():
        pltpu.sync_copy(acc.at[stripe], out_hbm.at[1, stripe])


@functools.cache
def _sc_edge():
    return pl.kernel(
        _sc_edge_body,
        out_type=jax.ShapeDtypeStruct((NCORE, NP, DD), jnp.float32),
        mesh=plsc.VectorSubcoreMesh(core_axis_name="c", subcore_axis_name="s",
                                    num_cores=NCORE, num_subcores=NSUB),
        scratch_types=[
            pltpu.VMEM_SHARED((NP, DD), jnp.float32),
            pltpu.VMEM((CHUNK,), jnp.int32),
            pltpu.VMEM((CHUNK,), jnp.int32),
            pltpu.VMEM((CHUNK, DD), jnp.float32),
            pltpu.SemaphoreType.DMA,
        ],
    )


def _tables_from(y):
    m = jnp.maximum(y, 0.0) + EPSM
    p = jnp.exp(m - SHIFT)
    q = m * p
    t0 = jnp.concatenate([q[:, :DD // 2], p[:, :DD // 2]], axis=1)
    t1 = jnp.concatenate([q[:, DD // 2:], p[:, DD // 2:]], axis=1)
    return jnp.stack([t0, t1], axis=0)


def _prep_body(x_ref, t_ref):
    t_ref[...] = _tables_from(x_ref[...])


_prep = pl.pallas_call(
    _prep_body,
    grid=(NBLK,),
    in_specs=[pl.BlockSpec((BLK, DD), lambda i: (i, 0))],
    out_specs=pl.BlockSpec((NCORE, BLK, DD), lambda i: (0, i, 0)),
    out_shape=jax.ShapeDtypeStruct((NCORE, NN, DD), jnp.float32),
)


def _mlp1_body(acc_ref, x_ref, w1_ref, b1_ref, h1_ref, st_ref):
    num = jnp.concatenate([acc_ref[0, :, :DD // 2], acc_ref[1, :, :DD // 2]],
                          axis=1)
    den = jnp.concatenate([acc_ref[0, :, DD // 2:], acc_ref[1, :, DD // 2:]],
                          axis=1)
    agg = num / (den + 1e-16)
    h = agg + x_ref[...]
    h1 = jnp.dot(h, w1_ref[...], preferred_element_type=jnp.float32)
    h1 = h1 + b1_ref[...]
    h1_ref[...] = h1
    blk = jnp.concatenate([jnp.sum(h1, axis=0, keepdims=True),
                           jnp.sum(h1 * h1, axis=0, keepdims=True)], axis=0)

    @pl.when(pl.program_id(0) == 0)
    def _():
        st_ref[...] = blk

    @pl.when(pl.program_id(0) != 0)
    def _():
        st_ref[...] += blk


_mlp1 = pl.pallas_call(
    _mlp1_body,
    grid=(NBLK,),
    in_specs=[
        pl.BlockSpec((NCORE, BLK, DD), lambda i: (0, i, 0)),
        pl.BlockSpec((BLK, DD), lambda i: (i, 0)),
        pl.BlockSpec((DD, HH), lambda i: (0, 0)),
        pl.BlockSpec((1, HH), lambda i: (0, 0)),
    ],
    out_specs=[
        pl.BlockSpec((BLK, HH), lambda i: (i, 0)),
        pl.BlockSpec((2, HH), lambda i: (0, 0)),
    ],
    out_shape=[
        jax.ShapeDtypeStruct((NN, HH), jnp.float32),
        jax.ShapeDtypeStruct((2, HH), jnp.float32),
    ],
)


def _mlp2_body(h1_ref, st_ref, g_ref, be_ref, w2_ref, b2_ref, y_ref, t_ref):
    mean = st_ref[0:1, :] * (1.0 / NN)
    ex2 = st_ref[1:2, :] * (1.0 / NN)
    var = ex2 - mean * mean
    h1n = g_ref[...] * (h1_ref[...] - mean) * lax.rsqrt(var + 1e-5) + be_ref[...]
    r = jnp.maximum(h1n, 0.0)
    y = jnp.dot(r, w2_ref[...], preferred_element_type=jnp.float32)
    y = y + b2_ref[...]
    y = jnp.where(y > 0.0, y, jnp.exp(jnp.minimum(y, 0.0)) - 1.0)
    y_ref[...] = y
    t_ref[...] = _tables_from(y)


_mlp2 = pl.pallas_call(
    _mlp2_body,
    grid=(NBLK,),
    in_specs=[
        pl.BlockSpec((BLK, HH), lambda i: (i, 0)),
        pl.BlockSpec((2, HH), lambda i: (0, 0)),
        pl.BlockSpec((1, HH), lambda i: (0, 0)),
        pl.BlockSpec((1, HH), lambda i: (0, 0)),
        pl.BlockSpec((HH, DD), lambda i: (0, 0)),
        pl.BlockSpec((1, DD), lambda i: (0, 0)),
    ],
    out_specs=[
        pl.BlockSpec((BLK, DD), lambda i: (i, 0)),
        pl.BlockSpec((NCORE, BLK, DD), lambda i: (0, i, 0)),
    ],
    out_shape=[
        jax.ShapeDtypeStruct((NN, DD), jnp.float32),
        jax.ShapeDtypeStruct((NCORE, NN, DD), jnp.float32),
    ],
)


def kernel(x, edge_index, W1_0, b1_0, gamma_0, beta_0, W2_0, b2_0,
           W1_1, b1_1, gamma_1, beta_1, W2_1, b2_1):
    src = edge_index[0].astype(jnp.int32)
    dst = edge_index[1].astype(jnp.int32)
    zeros = jnp.zeros((RPT, DD), jnp.float32)

    def layer(tables, xin, W1, b1, g, be, W2, b2):
        acc = _sc_edge()(tables[0], tables[1], src, dst, zeros)[:, :NN, :]
        h1, st = _mlp1(acc, xin, W1, b1.reshape(1, HH))
        y, tnext = _mlp2(h1, st, g.reshape(1, HH), be.reshape(1, HH),
                         W2, b2.reshape(1, DD))
        return y, tnext

    t = _prep(x)
    y0, t = layer(t, x, W1_0, b1_0, gamma_0, beta_0, W2_0, b2_0)
    y1, _ = layer(t, y0, W1_1, b1_1, gamma_1, beta_1, W2_1, b2_1)
    return y1


# speedup vs baseline: 7.3653x; 1.2097x over previous
import functools

import jax
import jax.numpy as jnp
from jax import lax
from jax.experimental import pallas as pl
from jax.experimental.pallas import tpu as pltpu
from jax.experimental.pallas import tpu_sc as plsc

NN = 10000
DD = 128
HH = 256
EE = 320000
EPSM = 1e-7
SHIFT = 30.0

NCORE = 2
NSUB = 16
EPT = EE // NSUB
CHUNK = 128
NCHUNK = 160
BIDX = 40
NIDX = NCHUNK // BIDX
EPTP = CHUNK * NCHUNK
NP = 10112
RPT = NP // NSUB
PADROW = NN

BLK = 2000
NBLK = NN // BLK


def _sc_edge_body(t0_hbm, t1_hbm, src_hbm, dst_hbm, zeros_hbm, out_hbm,
                  acc, srcs_v, dsts_v, rows0, rows1, sem0, sem1):
    c = lax.axis_index("c")
    s = lax.axis_index("s")
    stripe = pl.ds(s * RPT, RPT)

    pltpu.sync_copy(zeros_hbm, acc.at[stripe])
    plsc.subcore_barrier()

    def run_edges(tbl):
        def half(j, rows_b, sem_b):
            pltpu.make_async_copy(tbl.at[srcs_v.at[j]], rows_b, sem_b).wait()
            pltpu.sync_copy(rows_b, acc.at[dsts_v.at[j]], add=True)

            @pl.when(j + 2 < BIDX)
            def _():
                pltpu.async_copy(tbl.at[srcs_v.at[j + 2]], rows_b, sem_b)

        def body(k, carry):
            half(2 * k, rows0, sem0)
            half(2 * k + 1, rows1, sem1)
            return carry

        def block(b, carry):
            blk = pl.ds(b * BIDX, BIDX)
            pltpu.sync_copy(src_hbm.at[s, blk], srcs_v)
            pltpu.sync_copy(dst_hbm.at[s, blk], dsts_v)
            pltpu.async_copy(tbl.at[srcs_v.at[0]], rows0, sem0)
            pltpu.async_copy(tbl.at[srcs_v.at[1]], rows1, sem1)
            lax.fori_loop(0, BIDX // 2, body, 0)
            return carry

        lax.fori_loop(0, NIDX, block, 0)

    @pl.when(c == 0)
    def _():
        run_edges(t0_hbm)

    @pl.when(c == 1)
    def _():
        run_edges(t1_hbm)

    plsc.subcore_barrier()

    @pl.when(c == 0)
    def _():
        pltpu.sync_copy(acc.at[stripe], out_hbm.at[0, stripe])

    @pl.when(c == 1)
    def _():
        pltpu.sync_copy(acc.at[stripe], out_hbm.at[1, stripe])


@functools.cache
def _sc_edge():
    return pl.kernel(
        _sc_edge_body,
        out_type=jax.ShapeDtypeStruct((NCORE, NP, DD), jnp.float32),
        mesh=plsc.VectorSubcoreMesh(core_axis_name="c", subcore_axis_name="s",
                                    num_cores=NCORE, num_subcores=NSUB),
        scratch_types=[
            pltpu.VMEM_SHARED((NP, DD), jnp.float32),
            pltpu.VMEM((BIDX, CHUNK), jnp.int32),
            pltpu.VMEM((BIDX, CHUNK), jnp.int32),
            pltpu.VMEM((CHUNK, DD), jnp.float32),
            pltpu.VMEM((CHUNK, DD), jnp.float32),
            pltpu.SemaphoreType.DMA,
            pltpu.SemaphoreType.DMA,
        ],
    )


def _tables_from(y):
    m = jnp.maximum(y, 0.0) + EPSM
    p = jnp.exp(m - SHIFT)
    q = m * p
    t0 = jnp.concatenate([q[:, :DD // 2], p[:, :DD // 2]], axis=1)
    t1 = jnp.concatenate([q[:, DD // 2:], p[:, DD // 2:]], axis=1)
    return jnp.stack([t0, t1], axis=0)


def _prep_body(x_ref, t_ref):
    t_ref[...] = _tables_from(x_ref[...])


_prep = pl.pallas_call(
    _prep_body,
    grid=(NBLK,),
    in_specs=[pl.BlockSpec((BLK, DD), lambda i: (i, 0))],
    out_specs=pl.BlockSpec((NCORE, BLK, DD), lambda i: (0, i, 0)),
    out_shape=jax.ShapeDtypeStruct((NCORE, NN, DD), jnp.float32),
)


def _mlp1_body(acc_ref, x_ref, w1_ref, b1_ref, h1_ref, st_ref):
    num = jnp.concatenate([acc_ref[0, :, :DD // 2], acc_ref[1, :, :DD // 2]],
                          axis=1)
    den = jnp.concatenate([acc_ref[0, :, DD // 2:], acc_ref[1, :, DD // 2:]],
                          axis=1)
    agg = num / (den + 1e-16)
    h = agg + x_ref[...]
    h1 = jnp.dot(h, w1_ref[...], preferred_element_type=jnp.float32)
    h1 = h1 + b1_ref[...]
    h1_ref[...] = h1
    blk = jnp.concatenate([jnp.sum(h1, axis=0, keepdims=True),
                           jnp.sum(h1 * h1, axis=0, keepdims=True)], axis=0)

    @pl.when(pl.program_id(0) == 0)
    def _():
        st_ref[...] = blk

    @pl.when(pl.program_id(0) != 0)
    def _():
        st_ref[...] += blk


_mlp1 = pl.pallas_call(
    _mlp1_body,
    grid=(NBLK,),
    in_specs=[
        pl.BlockSpec((NCORE, BLK, DD), lambda i: (0, i, 0)),
        pl.BlockSpec((BLK, DD), lambda i: (i, 0)),
        pl.BlockSpec((DD, HH), lambda i: (0, 0)),
        pl.BlockSpec((1, HH), lambda i: (0, 0)),
    ],
    out_specs=[
        pl.BlockSpec((BLK, HH), lambda i: (i, 0)),
        pl.BlockSpec((2, HH), lambda i: (0, 0)),
    ],
    out_shape=[
        jax.ShapeDtypeStruct((NN, HH), jnp.float32),
        jax.ShapeDtypeStruct((2, HH), jnp.float32),
    ],
)


def _mlp2_body(h1_ref, st_ref, g_ref, be_ref, w2_ref, b2_ref, y_ref, t_ref):
    mean = st_ref[0:1, :] * (1.0 / NN)
    ex2 = st_ref[1:2, :] * (1.0 / NN)
    var = ex2 - mean * mean
    h1n = g_ref[...] * (h1_ref[...] - mean) * lax.rsqrt(var + 1e-5) + be_ref[...]
    r = jnp.maximum(h1n, 0.0)
    y = jnp.dot(r, w2_ref[...], preferred_element_type=jnp.float32)
    y = y + b2_ref[...]
    y = jnp.where(y > 0.0, y, jnp.exp(jnp.minimum(y, 0.0)) - 1.0)
    y_ref[...] = y
    t_ref[...] = _tables_from(y)


_mlp2 = pl.pallas_call(
    _mlp2_body,
    grid=(NBLK,),
    in_specs=[
        pl.BlockSpec((BLK, HH), lambda i: (i, 0)),
        pl.BlockSpec((2, HH), lambda i: (0, 0)),
        pl.BlockSpec((1, HH), lambda i: (0, 0)),
        pl.BlockSpec((1, HH), lambda i: (0, 0)),
        pl.BlockSpec((HH, DD), lambda i: (0, 0)),
        pl.BlockSpec((1, DD), lambda i: (0, 0)),
    ],
    out_specs=[
        pl.BlockSpec((BLK, DD), lambda i: (i, 0)),
        pl.BlockSpec((NCORE, BLK, DD), lambda i: (0, i, 0)),
    ],
    out_shape=[
        jax.ShapeDtypeStruct((NN, DD), jnp.float32),
        jax.ShapeDtypeStruct((NCORE, NN, DD), jnp.float32),
    ],
)


def kernel(x, edge_index, W1_0, b1_0, gamma_0, beta_0, W2_0, b2_0,
           W1_1, b1_1, gamma_1, beta_1, W2_1, b2_1):
    def tile_ids(flat, pad_value):
        per_tile = flat.astype(jnp.int32).reshape(NSUB, EPT)
        pad = jnp.full((NSUB, EPTP - EPT), pad_value, jnp.int32)
        return jnp.concatenate([per_tile, pad], axis=1).reshape(
            NSUB, NCHUNK, CHUNK)

    src = tile_ids(edge_index[0], 0)
    dst = tile_ids(edge_index[1], PADROW)
    zeros = jnp.zeros((RPT, DD), jnp.float32)

    def layer(tables, xin, W1, b1, g, be, W2, b2):
        acc = _sc_edge()(tables[0], tables[1], src, dst, zeros)[:, :NN, :]
        h1, st = _mlp1(acc, xin, W1, b1.reshape(1, HH))
        y, tnext = _mlp2(h1, st, g.reshape(1, HH), be.reshape(1, HH),
                         W2, b2.reshape(1, DD))
        return y, tnext

    t = _prep(x)
    y0, t = layer(t, x, W1_0, b1_0, gamma_0, beta_0, W2_0, b2_0)
    y1, _ = layer(t, y0, W1_1, b1_1, gamma_1, beta_1, W2_1, b2_1)
    return y1
